# baseline (speedup 1.0000x reference)
import jax
import jax.numpy as jnp
from jax import lax
from jax.experimental import pallas as pl

_MASK_CONST = -99999.0
_THRESHOLD = 0.1

Q = 1000
C = 21
P = 128 * 128
BLK = 1024
HB = BLK // 128
GRID = P // BLK


def _logits_body(logits_ref, ms_ref, cls_ref):
    logits = logits_ref[...]
    lmax = jnp.max(logits, axis=1, keepdims=True)
    e = jnp.exp(logits - lmax)
    s = jnp.sum(e, axis=1, keepdims=True)
    probs = e / s
    pmax = jnp.max(probs, axis=1, keepdims=True)
    citer = lax.broadcasted_iota(jnp.int32, (Q, C), 1)
    cls = jnp.min(jnp.where(probs == pmax, citer, jnp.int32(2**30)),
                  axis=1, keepdims=True)
    kp = (cls != (C - 1)) & (pmax > _THRESHOLD)
    ms_ref[...] = pmax * kp.astype(jnp.float32)
    cls_ref[...] = cls


def _masks_body(ms_ref, masks_ref, weighted_ref, mid_ref):
    ms = ms_ref[...]
    kf = (ms > 0.0).astype(jnp.float32)
    bf = (1.0 - kf) * _MASK_CONST

    x = masks_ref[...].reshape(Q, BLK)
    mx = jnp.max(x * kf + bf, axis=0, keepdims=True)
    ex = jnp.exp((x * kf + bf) - mx)
    denom = jnp.sum(ex, axis=0, keepdims=True)
    recip = 1.0 / denom
    weighted_ref[...] = ex * recip * ms
    mid_ref[...] = jnp.argmax(ex, axis=0, keepdims=True)


def kernel(pred_logits, pred_masks):
    ms, cls = pl.pallas_call(
        _logits_body,
        out_shape=[
            jax.ShapeDtypeStruct((Q, 1), jnp.float32),
            jax.ShapeDtypeStruct((Q, 1), jnp.int32),
        ],
    )(pred_logits)
    weighted, m_id = pl.pallas_call(
        _masks_body,
        grid=(GRID,),
        in_specs=[
            pl.BlockSpec((Q, 1), lambda i: (0, 0)),
            pl.BlockSpec((Q, HB, 128), lambda i: (0, i, 0)),
        ],
        out_specs=[
            pl.BlockSpec((Q, BLK), lambda i: (0, i)),
            pl.BlockSpec((1, BLK), lambda i: (0, i)),
        ],
        out_shape=[
            jax.ShapeDtypeStruct((Q, P), jnp.float32),
            jax.ShapeDtypeStruct((1, P), jnp.int32),
        ],
    )(ms, pred_masks)
    return (ms.reshape(Q), cls.reshape(Q), weighted, m_id.reshape(P))


# speedup vs baseline: 1.0378x; 1.0378x over previous
import jax
import jax.numpy as jnp
from jax import lax
from jax.experimental import pallas as pl

_MASK_CONST = -99999.0
_THRESHOLD = 0.1

Q = 1000
C = 21
P = 128 * 128
BLK = 2048
HB = BLK // 128
GRID = P // BLK


def _body(logits_ref, masks_ref, ms_ref, cls_ref, weighted_ref, mid_ref):
    @pl.when(pl.program_id(0) == 0)
    def _():
        logits = logits_ref[...]
        lmax = jnp.max(logits, axis=1, keepdims=True)
        e = jnp.exp(logits - lmax)
        s = jnp.sum(e, axis=1, keepdims=True)
        probs = e / s
        pmax = jnp.max(probs, axis=1, keepdims=True)
        citer = lax.broadcasted_iota(jnp.int32, (Q, C), 1)
        cls = jnp.min(jnp.where(probs == pmax, citer, jnp.int32(2**30)),
                      axis=1, keepdims=True)
        kp = (cls != (C - 1)) & (pmax > _THRESHOLD)
        ms_ref[...] = pmax * kp.astype(jnp.float32)
        cls_ref[...] = cls

    ms = ms_ref[...]
    kf = (ms > 0.0).astype(jnp.float32)
    bf = (1.0 - kf) * _MASK_CONST

    x = masks_ref[...].reshape(Q, BLK)
    mx = jnp.max(x * kf + bf, axis=0, keepdims=True)
    ex = jnp.exp((x * kf + bf) - mx)
    denom = jnp.sum(ex, axis=0, keepdims=True)
    recip = 1.0 / denom
    weighted_ref[...] = ex * recip * ms
    mid_ref[...] = jnp.argmax(ex, axis=0, keepdims=True)


def kernel(pred_logits, pred_masks):
    ms, cls, weighted, m_id = pl.pallas_call(
        _body,
        grid=(GRID,),
        in_specs=[
            pl.BlockSpec((Q, C), lambda i: (0, 0)),
            pl.BlockSpec((Q, HB, 128), lambda i: (0, i, 0)),
        ],
        out_specs=[
            pl.BlockSpec((Q, 1), lambda i: (0, 0)),
            pl.BlockSpec((Q, 1), lambda i: (0, 0)),
            pl.BlockSpec((Q, BLK), lambda i: (0, i)),
            pl.BlockSpec((1, BLK), lambda i: (0, i)),
        ],
        out_shape=[
            jax.ShapeDtypeStruct((Q, 1), jnp.float32),
            jax.ShapeDtypeStruct((Q, 1), jnp.int32),
            jax.ShapeDtypeStruct((Q, P), jnp.float32),
            jax.ShapeDtypeStruct((1, P), jnp.int32),
        ],
    )(pred_logits, pred_masks)
    return (ms.reshape(Q), cls.reshape(Q), weighted, m_id.reshape(P))


# speedup vs baseline: 1.0378x; 1.0000x over previous
import jax
import jax.numpy as jnp
from jax import lax
from jax.experimental import pallas as pl
from jax.experimental.pallas import tpu as pltpu

_MASK_CONST = -99999.0
_THRESHOLD = 0.1

Q = 1000
C = 21
P = 128 * 128
BLK = 2048
HB = BLK // 128
GRID = P // BLK


def _body(logits_ref, masks_ref, ms_ref, cls_ref, weighted_ref, mid_ref):
    @pl.when(pl.program_id(0) == 0)
    def _():
        logits = logits_ref[...]
        lmax = jnp.max(logits, axis=1, keepdims=True)
        e = jnp.exp(logits - lmax)
        s = jnp.sum(e, axis=1, keepdims=True)
        probs = e / s
        pmax = jnp.max(probs, axis=1, keepdims=True)
        citer = lax.broadcasted_iota(jnp.int32, (Q, C), 1)
        cls = jnp.min(jnp.where(probs == pmax, citer, jnp.int32(2**30)),
                      axis=1, keepdims=True)
        kp = (cls != (C - 1)) & (pmax > _THRESHOLD)
        ms_ref[...] = pmax * kp.astype(jnp.float32)
        cls_ref[...] = cls

    ms = ms_ref[...]
    kf = (ms > 0.0).astype(jnp.float32)
    bf = (1.0 - kf) * _MASK_CONST

    x = masks_ref[...].reshape(Q, BLK)
    mx = jnp.max(x * kf + bf, axis=0, keepdims=True)
    ex = jnp.exp((x * kf + bf) - mx)
    denom = jnp.sum(ex, axis=0, keepdims=True)
    recip = 1.0 / denom
    weighted_ref[...] = ex * recip * ms
    mid_ref[...] = jnp.argmax(ex, axis=0, keepdims=True)


def kernel(pred_logits, pred_masks):
    ms, cls, weighted, m_id = pl.pallas_call(
        _body,
        grid=(GRID,),
        compiler_params=pltpu.CompilerParams(
            dimension_semantics=("parallel",)),
        in_specs=[
            pl.BlockSpec((Q, C), lambda i: (0, 0)),
            pl.BlockSpec((Q, HB, 128), lambda i: (0, i, 0)),
        ],
        out_specs=[
            pl.BlockSpec((Q, 1), lambda i: (0, 0)),
            pl.BlockSpec((Q, 1), lambda i: (0, 0)),
            pl.BlockSpec((Q, BLK), lambda i: (0, i)),
            pl.BlockSpec((1, BLK), lambda i: (0, i)),
        ],
        out_shape=[
            jax.ShapeDtypeStruct((Q, 1), jnp.float32),
            jax.ShapeDtypeStruct((Q, 1), jnp.int32),
            jax.ShapeDtypeStruct((Q, P), jnp.float32),
            jax.ShapeDtypeStruct((1, P), jnp.int32),
        ],
    )(pred_logits, pred_masks)
    return (ms.reshape(Q), cls.reshape(Q), weighted, m_id.reshape(P))
